# baseline (speedup 1.0000x reference)
import functools

import numpy as np

import jax
import jax.numpy as jnp
from jax import lax
from jax.experimental import pallas as pl
from jax.experimental.pallas import tpu as pltpu

C_OUT = 10
OH = OW = 14
HW = OH * OW
IN_HW = 28 * 28
N_TAPS = 9
N_CLASSES = 10
ROW_STRIDE = 16
I_ROWS = C_OUT * ROW_STRIDE
K_CAT = OH * I_ROWS
M_PAD = 16
W3_K = 3 * 28


def _w3_selection():
    s = np.zeros((N_TAPS, ROW_STRIDE, W3_K), np.float32)
    for dr in range(3):
        for dj in range(3):
            t = dr * 3 + dj
            for j in range(OW):
                cc = 2 * j + dj - 1
                if not 0 <= cc < 28:
                    continue
                s[t, j, dr * 28 + cc] = 1.0
    return s


_W3_SEL = _w3_selection()


def _cnn_kernel(x_ref, w3_ref, cb_ref, fcw_ref, fcb_ref, o_ref,
                xf_ref, act_ref):
    nb = o_ref.shape[1]

    xf_ref[...] = x_ref[...].reshape(IN_HW, nb)

    crow = lax.broadcasted_iota(jnp.int32, (I_ROWS, 128), 0) // ROW_STRIDE
    bias = jnp.zeros((I_ROWS, 128), jnp.float32)
    for c in range(C_OUT):
        bias = jnp.where(crow == c, cb_ref[c], bias)
    bias = bias[:, 0:1]

    for i in range(OH):
        if i == 0:
            conv = jnp.dot(w3_ref[:, 28:], xf_ref[0:56, :],
                           preferred_element_type=jnp.float32)
        else:
            conv = jnp.dot(w3_ref[...],
                           xf_ref[pl.ds(28 * (2 * i - 1), W3_K), :],
                           preferred_element_type=jnp.float32)
        act_ref[pl.ds(i * I_ROWS, I_ROWS), :] = (
            jnp.maximum(conv + bias, 0.0))

    logits = jnp.dot(fcw_ref[...], act_ref[...],
                     preferred_element_type=jnp.float32) + fcb_ref[:, 0:1]

    row = lax.broadcasted_iota(jnp.int32, (M_PAD, nb), 0)
    valid = row < N_CLASSES
    masked = jnp.where(valid, logits, -jnp.inf)
    m = jnp.max(masked, axis=0, keepdims=True)
    e = jnp.exp(masked - m)
    lse = jnp.log(jnp.sum(e, axis=0, keepdims=True))
    o_ref[...] = jnp.where(valid, logits - m - lse, 0.0)


def _round_up(a, m):
    return ((a + m - 1) // m) * m


@jax.jit
def _forward(x, conv_w, conv_b, fc_w, fc_b):
    n = x.shape[0]
    xt = jnp.transpose(x.astype(jnp.float32), (1, 2, 3, 0)).reshape(IN_HW, n)

    nb = 2048
    n_pad = _round_up(max(n, 2 * nb), nb)
    if n_pad != n:
        xt = jnp.pad(xt, ((0, 0), (0, n_pad - n)))
    xt = xt.reshape(IN_HW, n_pad // 128, 128)

    cw = conv_w.astype(jnp.float32).reshape(C_OUT, N_TAPS)
    cb = conv_b.astype(jnp.float32)
    w3 = jnp.einsum('ct,tjk->cjk', cw,
                    jnp.asarray(_W3_SEL)).reshape(I_ROWS, W3_K)

    fcw_r = jnp.transpose(
        fc_w.astype(jnp.float32).reshape(N_CLASSES, C_OUT, OH, OW),
        (0, 2, 1, 3))
    fcw_p = jnp.pad(fcw_r, ((0, M_PAD - N_CLASSES), (0, 0), (0, 0),
                            (0, ROW_STRIDE - OW)))
    fcw_t = fcw_p.reshape(M_PAD, K_CAT)
    fcb_t = jnp.broadcast_to(
        jnp.pad(fc_b.astype(jnp.float32), (0, M_PAD - N_CLASSES))[:, None],
        (M_PAD, 128))

    grid = (n_pad // nb,)
    flops = n_pad * (2 * HW * N_TAPS * C_OUT + 2 * K_CAT * M_PAD)
    bytes_accessed = int(4 * (xt.size + w3.size + cb.size
                              + fcw_t.size + fcb_t.size + n_pad * M_PAD))

    out = pl.pallas_call(
        _cnn_kernel,
        out_shape=jax.ShapeDtypeStruct((M_PAD, n_pad), jnp.float32),
        grid=grid,
        in_specs=[
            pl.BlockSpec((IN_HW, nb // 128, 128), lambda i: (0, i, 0)),
            pl.BlockSpec((I_ROWS, W3_K), lambda i: (0, 0)),
            pl.BlockSpec(memory_space=pltpu.MemorySpace.SMEM),
            pl.BlockSpec((M_PAD, K_CAT), lambda i: (0, 0)),
            pl.BlockSpec((M_PAD, 128), lambda i: (0, 0)),
        ],
        out_specs=pl.BlockSpec((M_PAD, nb), lambda i: (0, i)),
        scratch_shapes=[pltpu.VMEM((IN_HW, nb), jnp.float32),
                        pltpu.VMEM((K_CAT, nb), jnp.float32)],
        compiler_params=pltpu.CompilerParams(
            dimension_semantics=("parallel",)),
        cost_estimate=pl.CostEstimate(
            flops=flops,
            transcendentals=n_pad * M_PAD,
            bytes_accessed=bytes_accessed),
    )(xt, w3, cb, fcw_t, fcb_t)

    return out[:N_CLASSES, :n].T


def kernel(x, conv_w, conv_b, fc_w, fc_b):
    return _forward(x, conv_w, conv_b, fc_w, fc_b)


# speedup vs baseline: 1.1357x; 1.1357x over previous
import functools

import numpy as np

import jax
import jax.numpy as jnp
from jax import lax
from jax.experimental import pallas as pl
from jax.experimental.pallas import tpu as pltpu

C_OUT = 10
OH = OW = 14
HW = OH * OW
IN_HW = 28 * 28
N_TAPS = 9
N_CLASSES = 10
ROW_STRIDE = 16
I_ROWS = C_OUT * ROW_STRIDE
K_CAT = OH * I_ROWS
M_PAD = 16
W3_K = 3 * 28


def _w3_selection():
    s = np.zeros((N_TAPS, ROW_STRIDE, W3_K), np.float32)
    for dr in range(3):
        for dj in range(3):
            t = dr * 3 + dj
            for j in range(OW):
                cc = 2 * j + dj - 1
                if not 0 <= cc < 28:
                    continue
                s[t, j, dr * 28 + cc] = 1.0
    return s


_W3_SEL = _w3_selection()


def _cnn_kernel(x_ref, w3_ref, cb_ref, fcw_ref, fcb_ref, o_ref,
                xf_ref, act_ref):
    nb = o_ref.shape[1]

    xf_ref[...] = x_ref[...].reshape(IN_HW, nb)

    crow = lax.broadcasted_iota(jnp.int32, (I_ROWS, 128), 0) // ROW_STRIDE
    bias = jnp.zeros((I_ROWS, 128), jnp.float32)
    for c in range(C_OUT):
        bias = jnp.where(crow == c, cb_ref[c], bias)
    bias = bias[:, 0:1]

    for i in range(OH):
        if i == 0:
            conv = jnp.dot(w3_ref[:, 28:], xf_ref[0:56, :],
                           preferred_element_type=jnp.float32)
        else:
            conv = jnp.dot(w3_ref[...],
                           xf_ref[pl.ds(28 * (2 * i - 1), W3_K), :],
                           preferred_element_type=jnp.float32)
        act_ref[pl.ds(i * I_ROWS, I_ROWS), :] = (
            jnp.maximum(conv + bias, 0.0))

    logits = jnp.dot(fcw_ref[...], act_ref[...],
                     preferred_element_type=jnp.float32) + fcb_ref[:, 0:1]

    row = lax.broadcasted_iota(jnp.int32, (M_PAD, nb), 0)
    valid = row < N_CLASSES
    masked = jnp.where(valid, logits, -jnp.inf)
    m = jnp.max(masked, axis=0, keepdims=True)
    e = jnp.exp(masked - m)
    lse = jnp.log(jnp.sum(e, axis=0, keepdims=True))
    o_ref[...] = jnp.where(valid, logits - m - lse, 0.0)


def _round_up(a, m):
    return ((a + m - 1) // m) * m


@jax.jit
def _forward(x, conv_w, conv_b, fc_w, fc_b):
    n = x.shape[0]
    xt = jnp.transpose(x.astype(jnp.float32), (1, 2, 3, 0)).reshape(IN_HW, n)

    nb = 1024
    n_pad = _round_up(max(n, 2 * nb), nb)
    if n_pad != n:
        xt = jnp.pad(xt, ((0, 0), (0, n_pad - n)))
    xt = xt.reshape(IN_HW, n_pad // 128, 128)

    cw = conv_w.astype(jnp.float32).reshape(C_OUT, N_TAPS)
    cb = conv_b.astype(jnp.float32)
    w3 = jnp.einsum('ct,tjk->cjk', cw,
                    jnp.asarray(_W3_SEL)).reshape(I_ROWS, W3_K)

    fcw_r = jnp.transpose(
        fc_w.astype(jnp.float32).reshape(N_CLASSES, C_OUT, OH, OW),
        (0, 2, 1, 3))
    fcw_p = jnp.pad(fcw_r, ((0, M_PAD - N_CLASSES), (0, 0), (0, 0),
                            (0, ROW_STRIDE - OW)))
    fcw_t = fcw_p.reshape(M_PAD, K_CAT)
    fcb_t = jnp.broadcast_to(
        jnp.pad(fc_b.astype(jnp.float32), (0, M_PAD - N_CLASSES))[:, None],
        (M_PAD, 128))

    grid = (n_pad // nb,)
    flops = n_pad * (2 * HW * N_TAPS * C_OUT + 2 * K_CAT * M_PAD)
    bytes_accessed = int(4 * (xt.size + w3.size + cb.size
                              + fcw_t.size + fcb_t.size + n_pad * M_PAD))

    out = pl.pallas_call(
        _cnn_kernel,
        out_shape=jax.ShapeDtypeStruct((M_PAD, n_pad), jnp.float32),
        grid=grid,
        in_specs=[
            pl.BlockSpec((IN_HW, nb // 128, 128), lambda i: (0, i, 0)),
            pl.BlockSpec((I_ROWS, W3_K), lambda i: (0, 0)),
            pl.BlockSpec(memory_space=pltpu.MemorySpace.SMEM),
            pl.BlockSpec((M_PAD, K_CAT), lambda i: (0, 0)),
            pl.BlockSpec((M_PAD, 128), lambda i: (0, 0)),
        ],
        out_specs=pl.BlockSpec((M_PAD, nb), lambda i: (0, i)),
        scratch_shapes=[pltpu.VMEM((IN_HW, nb), jnp.float32),
                        pltpu.VMEM((K_CAT, nb), jnp.float32)],
        compiler_params=pltpu.CompilerParams(
            dimension_semantics=("parallel",)),
        cost_estimate=pl.CostEstimate(
            flops=flops,
            transcendentals=n_pad * M_PAD,
            bytes_accessed=bytes_accessed),
    )(xt, w3, cb, fcw_t, fcb_t)

    return out[:N_CLASSES, :n].T


def kernel(x, conv_w, conv_b, fc_w, fc_b):
    return _forward(x, conv_w, conv_b, fc_w, fc_b)
